# baseline (speedup 1.0000x reference)
import functools

import jax
import jax.numpy as jnp
from jax import lax
from jax.experimental import pallas as pl
from jax.experimental.pallas import tpu as pltpu
from jax.experimental.pallas import tpu_sc as plsc

NC = 2
NS = 16
NWORK = NC * NS
L = 16


def _sc_scores(in_table, out_table, i_idx, o_idx, n_idx, B, E, W, N):
    b_per = B // NWORK
    n_ch = 2
    C = b_per // n_ch
    assert C % 8 == 0 and E == 2 * L

    mesh = plsc.VectorSubcoreMesh(core_axis_name="c", subcore_axis_name="s")

    @functools.partial(
        pl.kernel,
        out_type=(
            jax.ShapeDtypeStruct((B,), jnp.float32),
            jax.ShapeDtypeStruct((B,), jnp.float32),
        ),
        mesh=mesh,
        compiler_params=pltpu.CompilerParams(
            use_tc_tiling_on_sc=False, needs_layout_passes=False),
        scratch_types=[
            pltpu.VMEM((C,), jnp.int32),
            pltpu.VMEM((C * W,), jnp.int32),
            pltpu.VMEM((C * N,), jnp.int32),
            pltpu.VMEM((C, E), jnp.float32),
            pltpu.VMEM((C * W, E), jnp.float32),
            pltpu.VMEM((C * N, E), jnp.float32),
            pltpu.VMEM((C,), jnp.float32),
            pltpu.VMEM((C,), jnp.float32),
            pltpu.SemaphoreType.DMA,
        ],
    )
    def k(in_hbm, out_hbm, ii_hbm, oi_hbm, ni_hbm, spos_hbm, sneg_hbm,
          iv, ov, nv, irows, orows, nrows, sp, sn, sem):
        wid = lax.axis_index("s") * NC + lax.axis_index("c")
        for c in range(n_ch):
            base = wid * b_per + c * C
            pltpu.sync_copy(ii_hbm.at[pl.ds(base, C)], iv)
            pltpu.sync_copy(oi_hbm.at[pl.ds(base * W, C * W)], ov)
            pltpu.sync_copy(ni_hbm.at[pl.ds(base * N, C * N)], nv)
            cp_i = pltpu.async_copy(in_hbm.at[iv], irows, sem)
            cp_o = pltpu.async_copy(out_hbm.at[ov], orows, sem)
            cp_n = pltpu.async_copy(out_hbm.at[nv], nrows, sem)
            cp_i.wait()
            cp_o.wait()
            cp_n.wait()

            def body(g, _):
                bvec = g * L + lax.iota(jnp.int32, L)
                pacc = jnp.zeros((L,), jnp.float32)
                nacc = jnp.zeros((L,), jnp.float32)
                for e in range(E):
                    col = jnp.full((L,), e, jnp.int32)
                    v_in = plsc.load_gather(irows, [bvec, col])
                    pe = plsc.load_gather(orows, [bvec * W, col])
                    for w in range(1, W):
                        pe = pe + plsc.load_gather(orows, [bvec * W + w, col])
                    ne = plsc.load_gather(nrows, [bvec * N, col])
                    for n in range(1, N):
                        ne = ne + plsc.load_gather(nrows, [bvec * N + n, col])
                    pacc = pacc + v_in * pe
                    nacc = nacc + v_in * ne
                sp[pl.ds(g * L, L)] = pacc
                sn[pl.ds(g * L, L)] = nacc
                return 0

            lax.fori_loop(0, C // L, body, 0)
            pltpu.sync_copy(sp, spos_hbm.at[pl.ds(base, C)])
            pltpu.sync_copy(sn, sneg_hbm.at[pl.ds(base, C)])

    return k(in_table, out_table, i_idx, o_idx, n_idx)


def _tc_loss(s_pos, s_neg, B):
    def body(sp_ref, sn_ref, o_ref):
        def logsig(x):
            return jnp.minimum(x, 0.0) - jnp.log1p(jnp.exp(-jnp.abs(x)))

        o_ref[0, 0] = jnp.sum(logsig(sn_ref[...]) - logsig(sp_ref[...])) \
            * (1.0 / B)

    return pl.pallas_call(
        body,
        in_specs=[
            pl.BlockSpec(memory_space=pltpu.VMEM),
            pl.BlockSpec(memory_space=pltpu.VMEM),
        ],
        out_specs=pl.BlockSpec(memory_space=pltpu.SMEM),
        out_shape=jax.ShapeDtypeStruct((1, 1), jnp.float32),
    )(s_pos, s_neg)


def kernel(i, o, neg, in_table, out_table):
    B = i.shape[0]
    W = o.shape[1]
    N = neg.shape[1]
    E = in_table.shape[1]
    i32 = i.astype(jnp.int32)
    o32 = o.astype(jnp.int32).reshape(-1)
    n32 = neg.astype(jnp.int32).reshape(-1)
    s_pos, s_neg = _sc_scores(in_table, out_table, i32, o32, n32, B, E, W, N)
    loss = _tc_loss(s_pos.reshape(128, -1), s_neg.reshape(128, -1), B)
    return loss[0, 0]
